# baseline (speedup 1.0000x reference)
import functools

import jax
import jax.numpy as jnp
from jax import lax
from jax.experimental import pallas as pl
from jax.experimental.pallas import tpu as pltpu
from jax.experimental.pallas import tpu_sc as plsc

B = 16384
M = 1000000
D = 64
H = 524288
NBLK = 16384
G1 = H // NBLK
LAST_BLK = (M - 1) // NBLK

_INFO = plsc.get_sparse_core_info()
_NC = _INFO.num_cores
_NS = _INFO.num_subcores
_NW = _NC * _NS
_ROWS_PER_W = B // _NW
_CHUNK = 128
_NCHUNK = _ROWS_PER_W // _CHUNK


def _p2_body(t1_ref, t2_ref, w_ref, out_ref):
    w2 = w_ref[D:, :]
    ob = out_ref.bitcast(jnp.bfloat16)
    ob[:, :D] = lax.dot_general(
        t1_ref[...], w2, (((0,), (0,)), ((), ())),
        preferred_element_type=jnp.float32,
    ).astype(jnp.bfloat16)
    ob[:, D:] = lax.dot_general(
        t2_ref[...], w2, (((0,), (0,)), ((), ())),
        preferred_element_type=jnp.float32,
    ).astype(jnp.bfloat16)


def _transform(tT, W):
    return pl.pallas_call(
        _p2_body,
        grid=(G1,),
        in_specs=[
            pl.BlockSpec((D, NBLK), lambda g: (0, g)),
            pl.BlockSpec(
                (D, NBLK), lambda g: (0, jnp.minimum(g + G1, LAST_BLK))
            ),
            pl.BlockSpec((2 * D, D), lambda g: (0, 0)),
        ],
        out_specs=pl.BlockSpec((NBLK // 2, 2 * D), lambda g: (g, 0)),
        out_shape=jax.ShapeDtypeStruct((H // 2, 2 * D), jnp.uint32),
        compiler_params=pltpu.CompilerParams(
            dimension_semantics=("parallel",),
        ),
    )(tT, tT, W)


def _make_sc_gather():
    mesh = plsc.VectorSubcoreMesh(core_axis_name="c", subcore_axis_name="s")

    @functools.partial(
        pl.kernel,
        mesh=mesh,
        out_type=jax.ShapeDtypeStruct((B, 2 * D), jnp.uint32),
        scratch_types=[
            pltpu.VMEM((_ROWS_PER_W,), jnp.int32),
            pltpu.VMEM((_ROWS_PER_W, 2 * D), jnp.uint32),
            pltpu.SemaphoreType.DMA,
        ],
    )
    def gather_kernel(p2_hbm, idx_hbm, g2_hbm, idx_v, rows_v, sem):
        wid = lax.axis_index("s") * _NC + lax.axis_index("c")
        base = wid * _ROWS_PER_W
        pltpu.sync_copy(idx_hbm.at[pl.ds(base, _ROWS_PER_W)], idx_v)

        def fold(g, _):
            vec = idx_v[pl.ds(g * 16, 16)]
            idx_v[pl.ds(g * 16, 16)] = jnp.right_shift(
                jnp.bitwise_and(vec, H - 1), 1
            )
            return 0

        lax.fori_loop(0, _ROWS_PER_W // 16, fold, 0)

        copies = [
            pltpu.make_async_copy(
                p2_hbm.at[idx_v.at[pl.ds(c * _CHUNK, _CHUNK)]],
                rows_v.at[pl.ds(c * _CHUNK, _CHUNK)],
                sem,
            )
            for c in range(_NCHUNK)
        ]
        for cp in copies:
            cp.start()
        for cp in copies:
            cp.wait()
        pltpu.sync_copy(rows_v, g2_hbm.at[pl.ds(base, _ROWS_PER_W)])

    return gather_kernel


_sc_gather = _make_sc_gather()


_BLK = 2048


def _merge_body(old_ref, g2_ref, idx_ref, w_ref, b_ref, out_ref):
    g2 = lax.bitcast_convert_type(g2_ref[...], jnp.int32)
    i = idx_ref[...]
    w = jnp.where(i >= H, g2[:, D:], g2[:, :D])
    odd = jnp.bitwise_and(i, 1) != 0
    g = lax.bitcast_convert_type(
        jnp.where(
            odd,
            lax.bitwise_and(w, jnp.int32(-65536)),
            lax.shift_left(w, 16),
        ),
        jnp.float32,
    )
    out_ref[...] = (
        lax.dot_general(
            old_ref[...], w_ref[:D, :], (((1,), (0,)), ((), ())),
            preferred_element_type=jnp.float32,
        )
        + g
        + b_ref[...]
    )


def _merge(old_mems, g2, idxc, W, b2d):
    return pl.pallas_call(
        _merge_body,
        grid=(B // _BLK,),
        in_specs=[
            pl.BlockSpec((_BLK, D), lambda i: (i, 0)),
            pl.BlockSpec((_BLK, 2 * D), lambda i: (i, 0)),
            pl.BlockSpec((_BLK, 1), lambda i: (i, 0)),
            pl.BlockSpec((2 * D, D), lambda i: (0, 0)),
            pl.BlockSpec((1, D), lambda i: (0, 0)),
        ],
        out_specs=pl.BlockSpec((_BLK, D), lambda i: (i, 0)),
        out_shape=jax.ShapeDtypeStruct((B, D), jnp.float32),
        compiler_params=pltpu.CompilerParams(
            dimension_semantics=("parallel",),
        ),
    )(old_mems, g2, idxc, W, b2d)


def kernel(old_mems, new_mems, indices, W, b):
    idx = indices.astype(jnp.int32)
    p2 = _transform(new_mems.T, W)
    g2 = _sc_gather(p2, idx)
    return _merge(old_mems, g2, idx.reshape(B, 1), W, b.reshape(1, D))


# speedup vs baseline: 1.8062x; 1.8062x over previous
import functools

import jax
import jax.numpy as jnp
from jax import lax
from jax.experimental import pallas as pl
from jax.experimental.pallas import tpu as pltpu
from jax.experimental.pallas import tpu_sc as plsc

B = 16384
M = 1000000
D = 64
H = 524288
NBLK = 16384
G1 = H // NBLK
LAST_BLK = (M - 1) // NBLK

_INFO = plsc.get_sparse_core_info()
_NC = _INFO.num_cores
_NS = _INFO.num_subcores
_NW = _NC * _NS
_ROWS_PER_W = B // _NW
_CHUNK = 128
_NCHUNK = _ROWS_PER_W // _CHUNK


def _p2_body(t1_ref, t2_ref, w_ref, out_ref):
    w2 = w_ref[D:, :]
    out_ref[:, :D] = lax.dot_general(
        t1_ref[...], w2, (((0,), (0,)), ((), ())),
        preferred_element_type=jnp.float32,
    )
    out_ref[:, D:] = lax.dot_general(
        t2_ref[...], w2, (((0,), (0,)), ((), ())),
        preferred_element_type=jnp.float32,
    )


def _transform(tT, W):
    return pl.pallas_call(
        _p2_body,
        grid=(G1,),
        in_specs=[
            pl.BlockSpec((D, NBLK), lambda g: (0, g)),
            pl.BlockSpec(
                (D, NBLK), lambda g: (0, jnp.minimum(g + G1, LAST_BLK))
            ),
            pl.BlockSpec((2 * D, D), lambda g: (0, 0)),
        ],
        out_specs=pl.BlockSpec((NBLK, 2 * D), lambda g: (g, 0)),
        out_shape=jax.ShapeDtypeStruct((H, 2 * D), jnp.float32),
        compiler_params=pltpu.CompilerParams(
            dimension_semantics=("parallel",),
        ),
    )(tT, tT, W)


def _make_sc_gather():
    mesh = plsc.VectorSubcoreMesh(core_axis_name="c", subcore_axis_name="s")

    @functools.partial(
        pl.kernel,
        mesh=mesh,
        out_type=jax.ShapeDtypeStruct((B, 2 * D), jnp.float32),
        scratch_types=[
            pltpu.VMEM((_ROWS_PER_W,), jnp.int32),
            pltpu.VMEM((_ROWS_PER_W, 2 * D), jnp.float32),
            pltpu.SemaphoreType.DMA,
        ],
    )
    def gather_kernel(p2_hbm, idx_hbm, g2_hbm, idx_v, rows_v, sem):
        wid = lax.axis_index("s") * _NC + lax.axis_index("c")
        base = wid * _ROWS_PER_W
        pltpu.sync_copy(idx_hbm.at[pl.ds(base, _ROWS_PER_W)], idx_v)

        def fold(g, _):
            vec = idx_v[pl.ds(g * 16, 16)]
            idx_v[pl.ds(g * 16, 16)] = jnp.where(vec >= H, vec - H, vec)
            return 0

        lax.fori_loop(0, _ROWS_PER_W // 16, fold, 0)

        copies = [
            pltpu.make_async_copy(
                p2_hbm.at[idx_v.at[pl.ds(c * _CHUNK, _CHUNK)]],
                rows_v.at[pl.ds(c * _CHUNK, _CHUNK)],
                sem,
            )
            for c in range(_NCHUNK)
        ]
        for cp in copies:
            cp.start()
        for cp in copies:
            cp.wait()
        pltpu.sync_copy(rows_v, g2_hbm.at[pl.ds(base, _ROWS_PER_W)])

    return gather_kernel


_sc_gather = _make_sc_gather()


_BLK = 2048


def _merge_body(old_ref, g2_ref, idx_ref, w_ref, b_ref, out_ref):
    g2 = g2_ref[...]
    sel = idx_ref[...] >= H
    g = jnp.where(sel, g2[:, D:], g2[:, :D])
    out_ref[...] = (
        lax.dot_general(
            old_ref[...], w_ref[:D, :], (((1,), (0,)), ((), ())),
            preferred_element_type=jnp.float32,
        )
        + g
        + b_ref[...]
    )


def _merge(old_mems, g2, idxc, W, b2d):
    return pl.pallas_call(
        _merge_body,
        grid=(B // _BLK,),
        in_specs=[
            pl.BlockSpec((_BLK, D), lambda i: (i, 0)),
            pl.BlockSpec((_BLK, 2 * D), lambda i: (i, 0)),
            pl.BlockSpec((_BLK, 1), lambda i: (i, 0)),
            pl.BlockSpec((2 * D, D), lambda i: (0, 0)),
            pl.BlockSpec((1, D), lambda i: (0, 0)),
        ],
        out_specs=pl.BlockSpec((_BLK, D), lambda i: (i, 0)),
        out_shape=jax.ShapeDtypeStruct((B, D), jnp.float32),
        compiler_params=pltpu.CompilerParams(
            dimension_semantics=("parallel",),
        ),
    )(old_mems, g2, idxc, W, b2d)


def kernel(old_mems, new_mems, indices, W, b):
    idx = indices.astype(jnp.int32)
    p2 = _transform(new_mems.T, W)
    g2 = _sc_gather(p2, idx)
    return _merge(old_mems, g2, idx.reshape(B, 1), W, b.reshape(1, D))
